# Initial kernel scaffold; baseline (speedup 1.0000x reference)
import jax
import jax.numpy as jnp
from jax.experimental import pallas as pl


def kernel(x, edge_index, edge_attr, idx, W_msg, b_msg, W_upd, b_upd):
    raise NotImplementedError("write your pallas kernel here")


# speedup vs baseline: 2.7735x; 2.7735x over previous
import functools

import jax
import jax.numpy as jnp
from jax import lax
from jax.experimental import pallas as pl
from jax.experimental.pallas import tpu as pltpu
from jax.experimental.pallas import tpu_sc as plsc

N = 10000
E = 320000
D = 128
DE = 16
B = 1024

NC = 2
NS = 16
NW = NC * NS
EPW = E // NW
C = 80
NCHUNK = EPW // C
ZR = 125
RPT = N // NS
BPT = B // NS


def _xa_body(x_ref, w_ref, o_ref):
    o_ref[...] = jnp.dot(x_ref[...], w_ref[...],
                         preferred_element_type=jnp.float32)


def _eb_body(a_ref, w_ref, b_ref, o_ref):
    o_ref[...] = jnp.dot(a_ref[...], w_ref[...],
                         preferred_element_type=jnp.float32) + b_ref[...]


_EB_R = 8000


def _tc_pre(x, attr, W1, W2, b_msg):
    xa = pl.pallas_call(
        _xa_body,
        out_shape=jax.ShapeDtypeStruct((N, D), jnp.float32),
    )(x, W1)
    eb = pl.pallas_call(
        _eb_body,
        grid=(E // _EB_R,),
        in_specs=[
            pl.BlockSpec((_EB_R, DE), lambda i: (i, 0)),
            pl.BlockSpec((DE, D), lambda i: (0, 0)),
            pl.BlockSpec((1, D), lambda i: (0, 0)),
        ],
        out_specs=pl.BlockSpec((_EB_R, D), lambda i: (i, 0)),
        out_shape=jax.ShapeDtypeStruct((E, D), jnp.float32),
    )(attr, W2, b_msg.reshape(1, D))
    return xa, eb


def _sc_body(xa_hbm, eb_hbm, src_hbm, dst_hbm, idx_hbm, x_hbm, zeros_hbm,
             aggsel_hbm, xsel_hbm,
             agg_sh, src_v, dst_v, g_v, e_v, z_v, idx_v, sel_v, sem):
    cid = lax.axis_index("c")
    sid = lax.axis_index("s")
    wid = cid * NS + sid

    pltpu.sync_copy(zeros_hbm, z_v)
    for j in range(RPT // ZR):
        pltpu.sync_copy(z_v, agg_sh.at[pl.ds(sid * RPT + j * ZR, ZR)])
    plsc.subcore_barrier()

    def chunk(i, carry):
        base = wid * EPW + i * C
        pltpu.sync_copy(src_hbm.at[pl.ds(base, C)], src_v)
        pltpu.sync_copy(dst_hbm.at[pl.ds(base, C)], dst_v)
        pltpu.sync_copy(eb_hbm.at[pl.ds(base, C)], e_v)
        pltpu.async_copy(xa_hbm.at[src_v], g_v, sem).wait()

        def row(r, c2):
            for k in range(D // 16):
                s = pl.ds(k * 16, 16)
                e_v[r, s] = jnp.maximum(e_v[r, s] + g_v[r, s], 0.0)
            return c2

        lax.fori_loop(0, C, row, 0)
        pltpu.sync_copy(e_v, agg_sh.at[dst_v], add=True)
        return carry

    lax.fori_loop(0, NCHUNK, chunk, 0)
    plsc.subcore_barrier()

    pltpu.sync_copy(idx_hbm.at[pl.ds(sid * BPT, BPT)], idx_v)
    pltpu.async_copy(agg_sh.at[idx_v], sel_v, sem).wait()
    pltpu.sync_copy(sel_v, aggsel_hbm.at[cid, pl.ds(sid * BPT, BPT)])

    @pl.when(cid == 0)
    def _():
        pltpu.async_copy(x_hbm.at[idx_v], sel_v, sem).wait()
        pltpu.sync_copy(sel_v, xsel_hbm.at[pl.ds(sid * BPT, BPT)])


def _sc_call(xa, eb, src, dst, idx, x, zeros):
    mesh = plsc.VectorSubcoreMesh(core_axis_name="c", subcore_axis_name="s",
                                  num_cores=NC, num_subcores=NS)
    f = pl.kernel(
        _sc_body,
        out_type=(jax.ShapeDtypeStruct((NC, B, D), jnp.float32),
                  jax.ShapeDtypeStruct((B, D), jnp.float32)),
        mesh=mesh,
        scratch_types=[
            pltpu.VMEM_SHARED((N, D), jnp.float32),
            pltpu.VMEM((C,), jnp.int32),
            pltpu.VMEM((C,), jnp.int32),
            pltpu.VMEM((C, D), jnp.float32),
            pltpu.VMEM((C, D), jnp.float32),
            pltpu.VMEM((ZR, D), jnp.float32),
            pltpu.VMEM((BPT,), jnp.int32),
            pltpu.VMEM((BPT, D), jnp.float32),
            pltpu.SemaphoreType.DMA,
        ],
    )
    return f(xa, eb, src, dst, idx, x, zeros)


def _upd_body(xs_ref, a0_ref, a1_ref, w_ref, b_ref, o_ref):
    hcat = jnp.concatenate([xs_ref[...], a0_ref[...] + a1_ref[...]], axis=1)
    o_ref[...] = jnp.maximum(
        jnp.dot(hcat, w_ref[...], preferred_element_type=jnp.float32)
        + b_ref[...], 0.0)


def _tc_post(xsel, aggsel, W_upd, b_upd):
    return pl.pallas_call(
        _upd_body,
        out_shape=jax.ShapeDtypeStruct((B, D), jnp.float32),
    )(xsel, aggsel[0], aggsel[1], W_upd, b_upd.reshape(1, D))


def kernel(x, edge_index, edge_attr, idx, W_msg, b_msg, W_upd, b_upd):
    src = edge_index[0]
    dst = edge_index[1]
    W1 = W_msg[:D]
    W2 = W_msg[D:]
    xa, eb = _tc_pre(x, edge_attr, W1, W2, b_msg)
    zeros = jnp.zeros((ZR, D), jnp.float32)
    aggsel, xsel = _sc_call(xa, eb, src, dst, idx, x, zeros)
    return _tc_post(xsel, aggsel, W_upd, b_upd)


# speedup vs baseline: 4.4287x; 1.5968x over previous
import functools

import jax
import jax.numpy as jnp
from jax import lax
from jax.experimental import pallas as pl
from jax.experimental.pallas import tpu as pltpu
from jax.experimental.pallas import tpu_sc as plsc

N = 10000
E = 320000
D = 128
DE = 16
B = 1024

NC = 2
NS = 16
NW = NC * NS
EPW = E // NW
SUB = 80
SUBN = 1
C = SUB * SUBN
NCHUNK = EPW // C
RPT = N // NS
BPT = B // NS


def _xa_body(x_ref, w_ref, o_ref):
    o_ref[...] = jnp.dot(x_ref[...], w_ref[...],
                         preferred_element_type=jnp.float32)


def _eb_body(a_ref, w_ref, b_ref, o_ref):
    o_ref[...] = jnp.dot(a_ref[...], w_ref[...],
                         preferred_element_type=jnp.float32) + b_ref[...]


_EB_R = 8000


def _tc_pre(x, attr, W1, W2, b_msg):
    xa = pl.pallas_call(
        _xa_body,
        out_shape=jax.ShapeDtypeStruct((N, D), jnp.float32),
    )(x, W1)
    eb = pl.pallas_call(
        _eb_body,
        grid=(E // _EB_R,),
        in_specs=[
            pl.BlockSpec((_EB_R, DE), lambda i: (i, 0)),
            pl.BlockSpec((DE, D), lambda i: (0, 0)),
            pl.BlockSpec((1, D), lambda i: (0, 0)),
        ],
        out_specs=pl.BlockSpec((_EB_R, D), lambda i: (i, 0)),
        out_shape=jax.ShapeDtypeStruct((E, D), jnp.float32),
    )(attr, W2, b_msg.reshape(1, D))
    return xa, eb


def _sc_body(xa_hbm, eb_hbm, src_hbm, dst_hbm, idx_hbm, x_hbm, zeros_hbm,
             aggsel_hbm, xsel_hbm,
             agg_sh, src_v, dst_v, e_v, g_v, idx_v,
             sem_l0, sem_l1, sem_g0, sem_g1, sem_s0, sem_s1):
    cid = lax.axis_index("c")
    sid = lax.axis_index("s")
    wid = cid * NS + sid
    sem_l = (sem_l0, sem_l1)
    sem_g = (sem_g0, sem_g1)
    sem_s = (sem_s0, sem_s1)

    def issue_loads(k, s):
        crow = wid * NCHUNK + k
        ebase = wid * EPW + k * C
        pltpu.async_copy(src_hbm.at[crow], src_v.at[s], sem_l[s])
        pltpu.async_copy(dst_hbm.at[crow], dst_v.at[s], sem_l[s])
        pltpu.async_copy(eb_hbm.at[pl.ds(ebase, C)], e_v.at[s], sem_l[s])

    def wait_loads(s):
        pltpu.make_async_copy(src_hbm.at[0], src_v.at[s],
                              sem_l[s]).wait()
        pltpu.make_async_copy(dst_hbm.at[0], dst_v.at[s],
                              sem_l[s]).wait()
        pltpu.make_async_copy(eb_hbm.at[pl.ds(0, C)], e_v.at[s],
                              sem_l[s]).wait()

    def issue_gather(s):
        for j in range(SUBN):
            pltpu.async_copy(xa_hbm.at[src_v.at[s, j]],
                             g_v.at[s, pl.ds(j * SUB, SUB)], sem_g[s])

    def wait_gather(s):
        pltpu.make_async_copy(xa_hbm.at[pl.ds(0, C)], g_v.at[s],
                              sem_g[s]).wait()

    def issue_scatter(s):
        for j in range(SUBN):
            pltpu.async_copy(g_v.at[s, pl.ds(j * SUB, SUB)],
                             agg_sh.at[dst_v.at[s, j]], sem_s[s], add=True)

    def wait_scatter(s):
        pltpu.make_async_copy(eb_hbm.at[pl.ds(0, C)], e_v.at[s],
                              sem_s[s]).wait()

    issue_loads(0, 0)
    issue_loads(1, 1)
    pltpu.sync_copy(zeros_hbm, agg_sh.at[pl.ds(sid * RPT, RPT)])
    plsc.subcore_barrier()

    wait_loads(0)
    issue_gather(0)

    def pipeline_step(i, b):
        o = 1 - b

        @pl.when(i + 1 < NCHUNK)
        def _():
            wait_loads(o)

            @pl.when(i >= 1)
            def _():
                wait_scatter(o)

            issue_gather(o)

        wait_gather(b)

        @plsc.parallel_loop(0, C, step=1, unroll=4)
        def _(r):
            for k in range(D // 16):
                sl = pl.ds(k * 16, 16)
                g_v[b, r, sl] = jnp.maximum(g_v[b, r, sl] + e_v[b, r, sl], 0.0)

        issue_scatter(b)

        @pl.when(i + 2 < NCHUNK)
        def _():
            issue_loads(i + 2, b)

    def outer(t, carry):
        pipeline_step(2 * t, 0)
        pipeline_step(2 * t + 1, 1)
        return carry

    lax.fori_loop(0, NCHUNK // 2, outer, 0)
    pipeline_step(jnp.int32(NCHUNK - 1), 0)
    wait_scatter(0)
    wait_scatter(1)
    plsc.subcore_barrier()

    pltpu.sync_copy(idx_hbm.at[pl.ds(sid * BPT, BPT)], idx_v)
    pltpu.async_copy(agg_sh.at[idx_v], g_v.at[0, pl.ds(0, BPT)], sem_g0).wait()
    pltpu.sync_copy(g_v.at[0, pl.ds(0, BPT)],
                    aggsel_hbm.at[cid, pl.ds(sid * BPT, BPT)])

    @pl.when(cid == 0)
    def _():
        pltpu.async_copy(x_hbm.at[idx_v], g_v.at[1, pl.ds(0, BPT)],
                         sem_g1).wait()
        pltpu.sync_copy(g_v.at[1, pl.ds(0, BPT)],
                        xsel_hbm.at[pl.ds(sid * BPT, BPT)])


def _sc_call(xa, eb, src, dst, idx, x, zeros):
    mesh = plsc.VectorSubcoreMesh(core_axis_name="c", subcore_axis_name="s",
                                  num_cores=NC, num_subcores=NS)
    f = pl.kernel(
        _sc_body,
        out_type=(jax.ShapeDtypeStruct((NC, B, D), jnp.float32),
                  jax.ShapeDtypeStruct((B, D), jnp.float32)),
        mesh=mesh,
        scratch_types=[
            pltpu.VMEM_SHARED((N, D), jnp.float32),
            pltpu.VMEM((2, SUBN, SUB), jnp.int32),
            pltpu.VMEM((2, SUBN, SUB), jnp.int32),
            pltpu.VMEM((2, C, D), jnp.float32),
            pltpu.VMEM((2, C, D), jnp.float32),
            pltpu.VMEM((BPT,), jnp.int32),
            pltpu.SemaphoreType.DMA,
            pltpu.SemaphoreType.DMA,
            pltpu.SemaphoreType.DMA,
            pltpu.SemaphoreType.DMA,
            pltpu.SemaphoreType.DMA,
            pltpu.SemaphoreType.DMA,
        ],
    )
    return f(xa, eb, src, dst, idx, x, zeros)


def _upd_body(xs_ref, a0_ref, a1_ref, w_ref, b_ref, o_ref):
    hcat = jnp.concatenate([xs_ref[...], a0_ref[...] + a1_ref[...]], axis=1)
    o_ref[...] = jnp.maximum(
        jnp.dot(hcat, w_ref[...], preferred_element_type=jnp.float32)
        + b_ref[...], 0.0)


def _tc_post(xsel, aggsel, W_upd, b_upd):
    return pl.pallas_call(
        _upd_body,
        out_shape=jax.ShapeDtypeStruct((B, D), jnp.float32),
    )(xsel, aggsel[0], aggsel[1], W_upd, b_upd.reshape(1, D))


def kernel(x, edge_index, edge_attr, idx, W_msg, b_msg, W_upd, b_upd):
    src = edge_index[0].reshape(E // C, SUBN, SUB)
    dst = edge_index[1].reshape(E // C, SUBN, SUB)
    W1 = W_msg[:D]
    W2 = W_msg[D:]
    xa, eb = _tc_pre(x, edge_attr, W1, W2, b_msg)
    zeros = jnp.zeros((RPT, D), jnp.float32)
    aggsel, xsel = _sc_call(xa, eb, src, dst, idx, x, zeros)
    return _tc_post(xsel, aggsel, W_upd, b_upd)
